# baseline (speedup 1.0000x reference)
import jax
import jax.numpy as jnp
from jax import lax
from jax.experimental import pallas as pl
from jax.experimental.pallas import tpu as pltpu
from jax.experimental.pallas import tpu_sc as plsc

D = 32
B = 4096
L = 20
N = B * L
NC = 2
NS = 16
LANES = 16
NW = NC * NS
PW = N // NW
CW = B // NW
CHUNK = 128
NCHUNK = PW // CHUNK
NGRP = CHUNK // LANES


def _rsqrt(v):
    i = plsc.bitcast(v, jnp.int32)
    y = plsc.bitcast(jnp.full((LANES,), 0x5F3759DF, jnp.int32) - (i >> 1),
                     jnp.float32)
    for _ in range(3):
        y = y * (1.5 - 0.5 * v * y * y)
    return y


def _body(ctab, xtab, cidx_h, xidx_h, nidx_h, out_h,
          cidx_v, xidx_v, nidx_v, crow_v, xrow_v, nrow_v, acc_v,
          semx, semn):
    wid = lax.axis_index("s") * NC + lax.axis_index("c")

    pltpu.sync_copy(cidx_h.at[pl.ds(wid * CW, CW)], cidx_v)
    pltpu.sync_copy(ctab.at[cidx_v], crow_v)

    lanes = lax.iota(jnp.int32, LANES)

    def chunk_body(j, acc):
        base = wid * PW + j * CHUNK
        pltpu.sync_copy(xidx_h.at[pl.ds(base, CHUNK)], xidx_v)
        pltpu.sync_copy(nidx_h.at[pl.ds(base, CHUNK)], nidx_v)
        cx = pltpu.async_copy(xtab.at[xidx_v], xrow_v, semx)
        cn = pltpu.async_copy(xtab.at[nidx_v], nrow_v, semn)
        cx.wait()
        cn.wait()

        def grp_body(g, acc):
            rvec = g * LANES + lanes
            cvec = (j * CHUNK + rvec) // L
            z = jnp.zeros((LANES,), jnp.float32)
            dx, x2, dn, n2, c2 = z, z, z, z, z
            for d in range(D):
                dspl = jnp.full((LANES,), d, jnp.int32)
                x = plsc.load_gather(xrow_v, [rvec, dspl])
                nn = plsc.load_gather(nrow_v, [rvec, dspl])
                c = plsc.load_gather(crow_v, [cvec, dspl])
                dx = dx + c * x
                x2 = x2 + x * x
                dn = dn + c * nn
                n2 = n2 + nn * nn
                c2 = c2 + c * c
            inv_p = jnp.minimum(_rsqrt(c2 * x2), 1e8)
            inv_n = jnp.minimum(_rsqrt(c2 * n2), 1e8)
            return acc + jnp.maximum(dn * inv_n, 0.0) - dx * inv_p

        return lax.fori_loop(0, NGRP, grp_body, acc)

    acc = lax.fori_loop(0, NCHUNK, chunk_body,
                        jnp.zeros((LANES,), jnp.float32))
    acc_v[...] = acc
    pltpu.sync_copy(acc_v, out_h.at[pl.ds(wid * LANES, LANES)])


def kernel(centers_table, contexts_table, center_idxs, context_idxs, neg_idxs):
    cidx = center_idxs.astype(jnp.int32)
    xidx = context_idxs.reshape(N).astype(jnp.int32)
    nidx = neg_idxs.astype(jnp.int32)

    run = pl.kernel(
        _body,
        out_type=jax.ShapeDtypeStruct((NW * LANES,), jnp.float32),
        mesh=plsc.VectorSubcoreMesh(core_axis_name="c", subcore_axis_name="s"),
        compiler_params=pltpu.CompilerParams(
            use_tc_tiling_on_sc=False, needs_layout_passes=False),
        scratch_types=[
            pltpu.VMEM((CW,), jnp.int32),
            pltpu.VMEM((CHUNK,), jnp.int32),
            pltpu.VMEM((CHUNK,), jnp.int32),
            pltpu.VMEM((CW, D), jnp.float32),
            pltpu.VMEM((CHUNK, D), jnp.float32),
            pltpu.VMEM((CHUNK, D), jnp.float32),
            pltpu.VMEM((LANES,), jnp.float32),
            pltpu.SemaphoreType.DMA,
            pltpu.SemaphoreType.DMA,
        ],
    )
    part = run(centers_table, contexts_table, cidx, xidx, nidx)
    return jnp.float32(1.0) + jnp.sum(part) / jnp.float32(N)
